# baseline (speedup 1.0000x reference)
import jax
import jax.numpy as jnp
from jax import lax
from jax.experimental import pallas as pl
from jax.experimental.pallas import tpu as pltpu
from jax.experimental.pallas import tpu_sc as plsc

HIDDEN = 32
BATCH = 16384
NSEM = 8
VIEW_W = 128
PACK = VIEW_W // HIDDEN

_INFO = plsc.get_sparse_core_info()
NC = _INFO.num_cores
NS = _INFO.num_subcores
LANES = _INFO.num_lanes
NW = NC * NS
B_PER_W = BATCH // NW
PASS_IDS = 128
NPASS = B_PER_W // PASS_IDS
GRP_PER_PASS = PASS_IDS // LANES
IDX_CHUNK = 128
NCHUNK = B_PER_W // IDX_CHUNK


def _fetch_body(iid_hbm, it_hbm, out_hbm, iids_v, irows_v, *sems):
  wid = lax.axis_index("s") * NC + lax.axis_index("c")
  base = wid * B_PER_W

  pltpu.sync_copy(iid_hbm.at[pl.ds(base, B_PER_W)], iids_v)

  for p in range(NPASS):
    def fstep(g, carry, p=p):
      iidv = iids_v[pl.ds(p * PASS_IDS + g * LANES, LANES)]
      for k in range(LANES):
        pltpu.async_copy(it_hbm.at[pl.ds(iidv[k], 1)],
                         irows_v.at[pl.ds(g * LANES + k, 1)],
                         sems[k % NSEM])
      return carry

    lax.fori_loop(0, GRP_PER_PASS, fstep, 0)

    def wstep(_, carry):
      for j in range(NSEM):
        pltpu.make_async_copy(it_hbm.at[pl.ds(0, 1)],
                              irows_v.at[pl.ds(0, 1)], sems[j]).wait()
      return carry

    lax.fori_loop(0, GRP_PER_PASS * LANES // NSEM, wstep, 0)

    pltpu.sync_copy(irows_v, out_hbm.at[pl.ds(base + p * PASS_IDS, PASS_IDS)])


def _dot_body(uid_hbm, u128_hbm, irows_hbm, out_hbm,
              uids_v, uidx_v, urows_v, irows_v, out_v, sem):
  wid = lax.axis_index("s") * NC + lax.axis_index("c")
  base = wid * B_PER_W

  pltpu.sync_copy(uid_hbm.at[pl.ds(base, B_PER_W)], uids_v)

  for c in range(NCHUNK):
    for g in range(IDX_CHUNK // LANES):
      uidv = uids_v[pl.ds(c * IDX_CHUNK + g * LANES, LANES)]
      uidx_v[c, pl.ds(g * LANES, LANES)] = lax.shift_right_logical(uidv, 2)

  copies = []
  for c in range(NCHUNK):
    copies.append(pltpu.async_copy(
        u128_hbm.at[uidx_v.at[c]],
        urows_v.at[pl.ds(c * IDX_CHUNK, IDX_CHUNK)], sem))
  for cp in copies:
    cp.wait()

  for p in range(NPASS):
    pltpu.sync_copy(irows_hbm.at[pl.ds(base + p * PASS_IDS, PASS_IDS)],
                    irows_v)

    def cstep(g, carry, p=p):
      uidv = uids_v[pl.ds(p * PASS_IDS + g * LANES, LANES)]
      colbase = lax.bitwise_and(uidv, PACK - 1) * HIDDEN
      urow_ids = p * PASS_IDS + g * LANES + lax.iota(jnp.int32, LANES)
      irow_ids = g * LANES + lax.iota(jnp.int32, LANES)
      acc = jnp.zeros((LANES,), jnp.float32)
      for h in range(HIDDEN):
        hcol = jnp.full((LANES,), h, jnp.int32)
        uc = plsc.load_gather(urows_v, [urow_ids, colbase + h])
        ic = plsc.load_gather(irows_v, [irow_ids, hcol])
        acc = acc + uc * ic
      out_v[pl.ds(p * PASS_IDS + g * LANES, LANES)] = acc
      return carry

    lax.fori_loop(0, GRP_PER_PASS, cstep, 0)

  pltpu.sync_copy(out_v, out_hbm.at[pl.ds(base, B_PER_W)])


@jax.jit
def _mf(user_ids, item_ids, user_table, item_table):
  mesh = plsc.VectorSubcoreMesh(core_axis_name="c", subcore_axis_name="s")
  u128 = user_table.reshape(user_table.shape[0] // PACK, VIEW_W)

  fetch = pl.kernel(
      _fetch_body,
      mesh=mesh,
      out_type=jax.ShapeDtypeStruct((BATCH, HIDDEN), jnp.float32),
      scratch_types=[
          pltpu.VMEM((B_PER_W,), jnp.int32),
          pltpu.VMEM((PASS_IDS, HIDDEN), jnp.float32),
      ] + [pltpu.SemaphoreType.DMA] * NSEM,
      compiler_params=pltpu.CompilerParams(needs_layout_passes=False),
  )
  irows = fetch(item_ids, item_table)

  dot = pl.kernel(
      _dot_body,
      mesh=mesh,
      out_type=jax.ShapeDtypeStruct((BATCH,), jnp.float32),
      scratch_types=[
          pltpu.VMEM((B_PER_W,), jnp.int32),
          pltpu.VMEM((NCHUNK, IDX_CHUNK), jnp.int32),
          pltpu.VMEM((B_PER_W, VIEW_W), jnp.float32),
          pltpu.VMEM((PASS_IDS, HIDDEN), jnp.float32),
          pltpu.VMEM((B_PER_W,), jnp.float32),
          pltpu.SemaphoreType.DMA,
      ],
      compiler_params=pltpu.CompilerParams(needs_layout_passes=False),
  )
  return dot(user_ids, u128, irows)


def kernel(user_ids, item_ids, user_table, item_table):
  user_ids = user_ids.astype(jnp.int32)
  item_ids = item_ids.astype(jnp.int32)
  return _mf(user_ids, item_ids, user_table, item_table)


# speedup vs baseline: 1.0001x; 1.0001x over previous
import jax
import jax.numpy as jnp
from jax import lax
from jax.experimental import pallas as pl
from jax.experimental.pallas import tpu as pltpu
from jax.experimental.pallas import tpu_sc as plsc

HIDDEN = 32
BATCH = 16384
NSEM = 8
VIEW_W = 128
PACK = VIEW_W // HIDDEN

_INFO = plsc.get_sparse_core_info()
NC = _INFO.num_cores
NS = _INFO.num_subcores
LANES = _INFO.num_lanes
NW = NC * NS
B_PER_W = BATCH // NW
PASS_IDS = 128
NPASS = B_PER_W // PASS_IDS
GRP_PER_PASS = PASS_IDS // LANES
IDX_CHUNK = 128
NCHUNK = B_PER_W // IDX_CHUNK


def _fetch_body(iid_hbm, it_hbm, out_hbm, iids_v, irows_v, *sems):
  wid = lax.axis_index("s") * NC + lax.axis_index("c")
  base = wid * B_PER_W

  pltpu.sync_copy(iid_hbm.at[pl.ds(base, B_PER_W)], iids_v)

  for p in range(NPASS):
    def fstep(g, carry, p=p):
      iidv = iids_v[pl.ds(p * PASS_IDS + g * LANES, LANES)]
      for k in range(LANES):
        pltpu.async_copy(it_hbm.at[pl.ds(iidv[k], 1)],
                         irows_v.at[pl.ds(g * LANES + k, 1)],
                         sems[k % NSEM])
      return carry

    lax.fori_loop(0, GRP_PER_PASS, fstep, 0)

    def wstep(_, carry):
      for j in range(NSEM):
        pltpu.make_async_copy(it_hbm.at[pl.ds(0, 1)],
                              irows_v.at[pl.ds(0, 1)], sems[j]).wait()
      return carry

    lax.fori_loop(0, GRP_PER_PASS * LANES // NSEM, wstep, 0)

    pltpu.sync_copy(irows_v, out_hbm.at[pl.ds(base + p * PASS_IDS, PASS_IDS)])


def _dot_body(uid_hbm, u128_hbm, irows_hbm, out_hbm,
              uids_v, uidx_v, urows_v, irows_v, out_v, sem):
  wid = lax.axis_index("s") * NC + lax.axis_index("c")
  base = wid * B_PER_W

  pltpu.sync_copy(uid_hbm.at[pl.ds(base, B_PER_W)], uids_v)

  for c in range(NCHUNK):
    for g in range(IDX_CHUNK // LANES):
      uidv = uids_v[pl.ds(c * IDX_CHUNK + g * LANES, LANES)]
      uidx_v[c, pl.ds(g * LANES, LANES)] = lax.shift_right_logical(uidv, 2)

  copies = []
  for c in range(NCHUNK):
    copies.append(pltpu.async_copy(
        u128_hbm.at[uidx_v.at[c]],
        urows_v.at[pl.ds(c * IDX_CHUNK, IDX_CHUNK)], sem))
  for cp in copies:
    cp.wait()

  for p in range(NPASS):
    pltpu.sync_copy(irows_hbm.at[pl.ds(base + p * PASS_IDS, PASS_IDS)],
                    irows_v)

    def cstep(g, carry, p=p):
      uidv = uids_v[pl.ds(p * PASS_IDS + g * LANES, LANES)]
      colbase = lax.bitwise_and(uidv, PACK - 1) * HIDDEN
      urow_ids = p * PASS_IDS + g * LANES + lax.iota(jnp.int32, LANES)
      irow_ids = g * LANES + lax.iota(jnp.int32, LANES)
      acc = jnp.zeros((LANES,), jnp.float32)
      for h in range(HIDDEN):
        hcol = jnp.full((LANES,), h, jnp.int32)
        uc = plsc.load_gather(urows_v, [urow_ids, colbase + h])
        ic = plsc.load_gather(irows_v, [irow_ids, hcol])
        acc = acc + uc * ic
      out_v[pl.ds(p * PASS_IDS + g * LANES, LANES)] = acc
      return carry

    lax.fori_loop(0, GRP_PER_PASS, cstep, 0)

  pltpu.sync_copy(out_v, out_hbm.at[pl.ds(base, B_PER_W)])


@jax.jit
def _mf(user_ids, item_ids, user_table, item_table):
  mesh = plsc.VectorSubcoreMesh(core_axis_name="c", subcore_axis_name="s")
  one = (user_ids[0] * 0 + 1).astype(jnp.float32)
  u128 = user_table.reshape(user_table.shape[0] // PACK, VIEW_W) * one

  fetch = pl.kernel(
      _fetch_body,
      mesh=mesh,
      out_type=jax.ShapeDtypeStruct((BATCH, HIDDEN), jnp.float32),
      scratch_types=[
          pltpu.VMEM((B_PER_W,), jnp.int32),
          pltpu.VMEM((PASS_IDS, HIDDEN), jnp.float32),
      ] + [pltpu.SemaphoreType.DMA] * NSEM,
      compiler_params=pltpu.CompilerParams(needs_layout_passes=False),
  )
  irows = fetch(item_ids, item_table)

  dot = pl.kernel(
      _dot_body,
      mesh=mesh,
      out_type=jax.ShapeDtypeStruct((BATCH,), jnp.float32),
      scratch_types=[
          pltpu.VMEM((B_PER_W,), jnp.int32),
          pltpu.VMEM((NCHUNK, IDX_CHUNK), jnp.int32),
          pltpu.VMEM((B_PER_W, VIEW_W), jnp.float32),
          pltpu.VMEM((PASS_IDS, HIDDEN), jnp.float32),
          pltpu.VMEM((B_PER_W,), jnp.float32),
          pltpu.SemaphoreType.DMA,
      ],
      compiler_params=pltpu.CompilerParams(needs_layout_passes=False),
  )
  return dot(user_ids, u128, irows)


def kernel(user_ids, item_ids, user_table, item_table):
  user_ids = user_ids.astype(jnp.int32)
  item_ids = item_ids.astype(jnp.int32)
  return _mf(user_ids, item_ids, user_table, item_table)


# speedup vs baseline: 1.1782x; 1.1781x over previous
import jax
import jax.numpy as jnp
from jax import lax
from jax.experimental import pallas as pl
from jax.experimental.pallas import tpu as pltpu
from jax.experimental.pallas import tpu_sc as plsc

HIDDEN = 32
BATCH = 16384
NSEM = 8

_INFO = plsc.get_sparse_core_info()
NC = _INFO.num_cores
NS = _INFO.num_subcores
LANES = _INFO.num_lanes
NW = NC * NS
B_PER_W = BATCH // NW
PASS_IDS = 128
NPASS = B_PER_W // PASS_IDS
GRP_PER_PASS = PASS_IDS // LANES


def _mf_body(uid_hbm, iid_hbm, ut_hbm, it_hbm, out_hbm,
             uids_v, iids_v, urows_v, irows_v, out_v, *sems):
  wid = lax.axis_index("s") * NC + lax.axis_index("c")
  base = wid * B_PER_W

  pltpu.sync_copy(uid_hbm.at[pl.ds(base, B_PER_W)], uids_v)
  pltpu.sync_copy(iid_hbm.at[pl.ds(base, B_PER_W)], iids_v)

  def fire_group(p, g):
    uidv = uids_v[pl.ds(p * PASS_IDS + g * LANES, LANES)]
    iidv = iids_v[pl.ds(p * PASS_IDS + g * LANES, LANES)]
    for k in range(LANES):
      slot = g * LANES + k
      pltpu.async_copy(ut_hbm.at[pl.ds(uidv[k], 1)],
                       urows_v.at[pl.ds(slot, 1)], sems[k % NSEM])
      pltpu.async_copy(it_hbm.at[pl.ds(iidv[k], 1)],
                       irows_v.at[pl.ds(slot, 1)], sems[(k + NSEM // 2) % NSEM])

  def compute_group(p, g):
    rows = g * LANES + lax.iota(jnp.int32, LANES)
    acc = jnp.zeros((LANES,), jnp.float32)
    for h in range(HIDDEN):
      hcol = jnp.full((LANES,), h, jnp.int32)
      uc = plsc.load_gather(urows_v, [rows, hcol])
      ic = plsc.load_gather(irows_v, [rows, hcol])
      acc = acc + uc * ic
    out_v[pl.ds(p * PASS_IDS + g * LANES, LANES)] = acc

  for p in range(NPASS):
    def fstep(g, carry, p=p):
      fire_group(p, g)
      return carry

    lax.fori_loop(0, GRP_PER_PASS, fstep, 0)

    def wstep(_, carry):
      for j in range(NSEM):
        pltpu.make_async_copy(ut_hbm.at[pl.ds(0, 1)],
                              urows_v.at[pl.ds(0, 1)], sems[j]).wait()
      return carry

    lax.fori_loop(0, GRP_PER_PASS * 2 * LANES // NSEM, wstep, 0)

    def cstep(g, carry, p=p):
      compute_group(p, g)
      return carry

    lax.fori_loop(0, GRP_PER_PASS, cstep, 0)

  pltpu.sync_copy(out_v, out_hbm.at[pl.ds(base, B_PER_W)])


@jax.jit
def _mf(user_ids, item_ids, user_table, item_table):
  mesh = plsc.VectorSubcoreMesh(core_axis_name="c", subcore_axis_name="s")
  kern = pl.kernel(
      _mf_body,
      mesh=mesh,
      out_type=jax.ShapeDtypeStruct((BATCH,), jnp.float32),
      scratch_types=[
          pltpu.VMEM((B_PER_W,), jnp.int32),
          pltpu.VMEM((B_PER_W,), jnp.int32),
          pltpu.VMEM((PASS_IDS, HIDDEN), jnp.float32),
          pltpu.VMEM((PASS_IDS, HIDDEN), jnp.float32),
          pltpu.VMEM((B_PER_W,), jnp.float32),
      ] + [pltpu.SemaphoreType.DMA] * NSEM,
      compiler_params=pltpu.CompilerParams(needs_layout_passes=False),
  )
  return kern(user_ids, item_ids, user_table, item_table)


def kernel(user_ids, item_ids, user_table, item_table):
  user_ids = user_ids.astype(jnp.int32)
  item_ids = item_ids.astype(jnp.int32)
  return _mf(user_ids, item_ids, user_table, item_table)
